# Initial kernel scaffold; baseline (speedup 1.0000x reference)
import jax
import jax.numpy as jnp
from jax.experimental import pallas as pl


def kernel(x, edge_index, batch, W1, b1, g1, be1, L0_W, L0_b, Wc, bc, gc, bec, L1_W, L1_b):
    raise NotImplementedError("write your pallas kernel here")


# speedup vs baseline: 14.9113x; 14.9113x over previous
import functools

import jax
import jax.numpy as jnp
from jax import lax
from jax.experimental import pallas as pl
from jax.experimental.pallas import tpu as pltpu
from jax.experimental.pallas import tpu_sc as plsc

N = 10000
E = 320000
D = 128
HP = 8
G = 16

NC = 2
NS = 16
NW = NC * NS
EPW = E // NW
WIN = 128
NFULL = EPW // WIN
REM = EPW - NFULL * WIN
NROW = 10240
RPT = NROW // NS


def _first_layer_body(x_ref, w_ref, b_ref, g_ref, be_ref, h_ref):
    y = jnp.dot(x_ref[...], w_ref[...], preferred_element_type=jnp.float32)
    y = y + b_ref[...]
    mu = jnp.mean(y, axis=0, keepdims=True)
    var = jnp.mean((y - mu) ** 2, axis=0, keepdims=True)
    hn = g_ref[...] * (y - mu) * lax.rsqrt(var + 1e-5) + be_ref[...]
    h_ref[...] = jnp.maximum(hn, 0.0)


def _segment_sum_body(h_hbm, z_hbm, src_hbm, dst_hbm, out_hbm,
                      h_sh, agg_sh, stage_v, src_v, dst_v,
                      srcw, dstw, rows_v, srcw_r, dstw_r, rows_r, sem):
    cid = lax.axis_index("c")
    sid = lax.axis_index("s")
    wid = sid * NC + cid
    rbase = sid * RPT

    pltpu.sync_copy(h_hbm.at[pl.ds(rbase, RPT)], stage_v)
    pltpu.sync_copy(stage_v, h_sh.at[pl.ds(rbase, RPT)])
    pltpu.sync_copy(z_hbm.at[pl.ds(rbase, RPT)], stage_v)
    pltpu.sync_copy(stage_v, agg_sh.at[pl.ds(rbase, RPT)])

    ebase = wid * EPW
    pltpu.sync_copy(src_hbm.at[pl.ds(ebase, EPW)], src_v)
    pltpu.sync_copy(dst_hbm.at[pl.ds(ebase, EPW)], dst_v)
    plsc.subcore_barrier()

    def window(j, carry):
        off = j * WIN
        for k in range(WIN // 16):
            srcw[pl.ds(k * 16, 16)] = src_v[pl.ds(off + k * 16, 16)]
            dstw[pl.ds(k * 16, 16)] = dst_v[pl.ds(off + k * 16, 16)]
        pltpu.async_copy(h_sh.at[srcw], rows_v, sem).wait()
        pltpu.sync_copy(rows_v, agg_sh.at[dstw], add=True)
        return carry

    lax.fori_loop(0, NFULL, window, 0)

    roff = NFULL * WIN
    srcw_r[...] = src_v[pl.ds(roff, REM)]
    dstw_r[...] = dst_v[pl.ds(roff, REM)]
    pltpu.async_copy(h_sh.at[srcw_r], rows_r, sem).wait()
    pltpu.sync_copy(rows_r, agg_sh.at[dstw_r], add=True)

    plsc.subcore_barrier()
    pltpu.sync_copy(agg_sh.at[pl.ds(rbase, RPT)], stage_v)
    pltpu.sync_copy(stage_v, out_hbm.at[cid, pl.ds(rbase, RPT)])


_segment_sum_sc = functools.partial(
    pl.kernel,
    mesh=plsc.VectorSubcoreMesh(
        core_axis_name="c", subcore_axis_name="s",
        num_cores=NC, num_subcores=NS),
    out_type=jax.ShapeDtypeStruct((NC, NROW, HP), jnp.float32),
    compiler_params=pltpu.CompilerParams(use_tc_tiling_on_sc=False),
    scratch_types=[
        pltpu.VMEM_SHARED((NROW, HP), jnp.float32),
        pltpu.VMEM_SHARED((NROW, HP), jnp.float32),
        pltpu.VMEM((RPT, HP), jnp.float32),
        pltpu.VMEM((EPW,), jnp.int32),
        pltpu.VMEM((EPW,), jnp.int32),
        pltpu.VMEM((WIN,), jnp.int32),
        pltpu.VMEM((WIN,), jnp.int32),
        pltpu.VMEM((WIN, HP), jnp.float32),
        pltpu.VMEM((REM,), jnp.int32),
        pltpu.VMEM((REM,), jnp.int32),
        pltpu.VMEM((REM, HP), jnp.float32),
        pltpu.SemaphoreType.DMA,
    ],
)(_segment_sum_body)


def _second_layer_body(h_ref, a0_ref, a1_ref, batch_ref,
                       wc_ref, bc_ref, gc_ref, bec_ref,
                       l0_ref, l0b_ref, l1_ref, l1b_ref,
                       wsi_ref, node_ref):
    h = h_ref[...]
    np0 = jnp.sum(h * l0_ref[...], axis=1, keepdims=True) + l0b_ref[...]
    m = h + a0_ref[...] + a1_ref[...]
    y = jnp.dot(m, wc_ref[...], preferred_element_type=jnp.float32)
    y = y + bc_ref[...]
    mu = jnp.mean(y, axis=0, keepdims=True)
    var = jnp.mean((y - mu) ** 2, axis=0, keepdims=True)
    h2 = gc_ref[...] * (y - mu) * lax.rsqrt(var + 1e-5) + bec_ref[...]
    h2 = jnp.maximum(h2, 0.0)
    np1 = jnp.sum(h2 * l1_ref[...], axis=1, keepdims=True) + l1b_ref[...]
    node_ref[...] = np0 + np1
    ids = lax.broadcasted_iota(jnp.int32, (1, G), 1)
    mask = batch_ref[...] == ids
    neg = jnp.float32(-jnp.inf)
    w0 = jnp.max(jnp.where(mask, np0, neg), axis=0)
    w1 = jnp.max(jnp.where(mask, np1, neg), axis=0)
    wsi_ref[...] = (w0 + w1)[:, None]


def kernel(x, edge_index, batch, W1, b1, g1, be1, L0_W, L0_b,
           Wc, bc, gc, bec, L1_W, L1_b):
    f32 = jnp.float32
    W1p = jnp.pad(W1, ((0, 0), (0, HP - W1.shape[1])))
    b1p = jnp.pad(b1, (0, HP - b1.shape[0]))[None, :]
    g1p = jnp.pad(g1, (0, HP - g1.shape[0]))[None, :]
    be1p = jnp.pad(be1, (0, HP - be1.shape[0]))[None, :]
    Wcp = jnp.pad(Wc, ((0, HP - Wc.shape[0]), (0, HP - Wc.shape[1])))
    bcp = jnp.pad(bc, (0, HP - bc.shape[0]))[None, :]
    gcp = jnp.pad(gc, (0, HP - gc.shape[0]))[None, :]
    becp = jnp.pad(bec, (0, HP - bec.shape[0]))[None, :]
    l0 = jnp.pad(L0_W[:, 0], (0, HP - L0_W.shape[0]))[None, :]
    l1 = jnp.pad(L1_W[:, 0], (0, HP - L1_W.shape[0]))[None, :]
    l0b = L0_b.reshape(1, 1).astype(f32)
    l1b = L1_b.reshape(1, 1).astype(f32)

    h = pl.pallas_call(
        _first_layer_body,
        out_shape=jax.ShapeDtypeStruct((N, HP), f32),
    )(x, W1p, b1p, g1p, be1p)

    src = edge_index[0]
    dst = edge_index[1]
    hp = jnp.pad(h, ((0, NROW - N), (0, 0)))
    zeros = jnp.zeros((NROW, HP), f32)
    agg = _segment_sum_sc(hp, zeros, src, dst)

    wsi, node = pl.pallas_call(
        _second_layer_body,
        out_shape=(
            jax.ShapeDtypeStruct((G, 1), f32),
            jax.ShapeDtypeStruct((N, 1), f32),
        ),
    )(h, agg[0, :N], agg[1, :N], batch[:, None], Wcp, bcp, gcp, becp,
      l0, l0b, l1, l1b)
    return (wsi, node)


# speedup vs baseline: 16.6264x; 1.1150x over previous
import functools

import jax
import jax.numpy as jnp
from jax import lax
from jax.experimental import pallas as pl
from jax.experimental.pallas import tpu as pltpu
from jax.experimental.pallas import tpu_sc as plsc

N = 10000
E = 320000
D = 128
HP = 8
G = 16

NC = 2
NS = 16
NW = NC * NS
EPW = E // NW
WIN = 128
NFULL = EPW // WIN
REM = EPW - NFULL * WIN
NROW = 10240
RPT = NROW // NS


def _first_layer_body(x_ref, w_ref, b_ref, g_ref, be_ref, h_ref):
    y = jnp.dot(x_ref[...], w_ref[...], preferred_element_type=jnp.float32)
    y = y + b_ref[...]
    mu = jnp.mean(y, axis=0, keepdims=True)
    var = jnp.mean((y - mu) ** 2, axis=0, keepdims=True)
    hn = g_ref[...] * (y - mu) * lax.rsqrt(var + 1e-5) + be_ref[...]
    h_ref[...] = jnp.maximum(hn, 0.0)


def _segment_sum_body(h_hbm, z_hbm, src_hbm, dst_hbm, out_hbm,
                      h_sh, agg_sh, stage_v, src_v, dst_v,
                      rows_v, rows_v2, rows_r, sem, sem2):
    cid = lax.axis_index("c")
    sid = lax.axis_index("s")
    wid = sid * NC + cid
    rbase = sid * RPT

    pltpu.sync_copy(h_hbm.at[pl.ds(rbase, RPT)], stage_v)
    pltpu.sync_copy(stage_v, h_sh.at[pl.ds(rbase, RPT)])
    pltpu.sync_copy(z_hbm.at[pl.ds(rbase, RPT)], stage_v)
    pltpu.sync_copy(stage_v, agg_sh.at[pl.ds(rbase, RPT)])

    ebase = wid * EPW
    pltpu.sync_copy(src_hbm.at[pl.ds(ebase, EPW)], src_v)
    pltpu.sync_copy(dst_hbm.at[pl.ds(ebase, EPW)], dst_v)
    plsc.subcore_barrier()

    rows = (rows_v, rows_v2)
    sems = (sem, sem2)
    pltpu.async_copy(h_sh.at[src_v.at[pl.ds(0, WIN)]], rows[0], sems[0])

    def window(j, carry):
        off = j * WIN
        cur = jax.lax.rem(j, 2)
        nxt = 1 - cur
        noff = off + WIN

        @pl.when(j + 1 < NFULL)
        def _():
            for b in range(2):
                @pl.when(nxt == b)
                def _():
                    pltpu.async_copy(
                        h_sh.at[src_v.at[pl.ds(noff, WIN)]], rows[b], sems[b])

        for b in range(2):
            @pl.when(cur == b)
            def _():
                pltpu.make_async_copy(
                    h_sh.at[src_v.at[pl.ds(off, WIN)]], rows[b], sems[b]).wait()
                pltpu.sync_copy(rows[b], agg_sh.at[dst_v.at[pl.ds(off, WIN)]],
                                add=True)
        return carry

    lax.fori_loop(0, NFULL, window, 0)

    roff = NFULL * WIN
    pltpu.async_copy(h_sh.at[src_v.at[pl.ds(roff, REM)]], rows_r, sem).wait()
    pltpu.sync_copy(rows_r, agg_sh.at[dst_v.at[pl.ds(roff, REM)]], add=True)

    plsc.subcore_barrier()
    pltpu.sync_copy(agg_sh.at[pl.ds(rbase, RPT)], stage_v)
    pltpu.sync_copy(stage_v, out_hbm.at[cid, pl.ds(rbase, RPT)])


_segment_sum_sc = functools.partial(
    pl.kernel,
    mesh=plsc.VectorSubcoreMesh(
        core_axis_name="c", subcore_axis_name="s",
        num_cores=NC, num_subcores=NS),
    out_type=jax.ShapeDtypeStruct((NC, NROW, HP), jnp.float32),
    compiler_params=pltpu.CompilerParams(use_tc_tiling_on_sc=False),
    scratch_types=[
        pltpu.VMEM_SHARED((NROW, HP), jnp.float32),
        pltpu.VMEM_SHARED((NROW, HP), jnp.float32),
        pltpu.VMEM((RPT, HP), jnp.float32),
        pltpu.VMEM((EPW,), jnp.int32),
        pltpu.VMEM((EPW,), jnp.int32),
        pltpu.VMEM((WIN, HP), jnp.float32),
        pltpu.VMEM((WIN, HP), jnp.float32),
        pltpu.VMEM((REM, HP), jnp.float32),
        pltpu.SemaphoreType.DMA,
        pltpu.SemaphoreType.DMA,
    ],
)(_segment_sum_body)


def _second_layer_body(h_ref, a0_ref, a1_ref, batch_ref,
                       wc_ref, bc_ref, gc_ref, bec_ref,
                       l0_ref, l0b_ref, l1_ref, l1b_ref,
                       wsi_ref, node_ref):
    h = h_ref[...]
    np0 = jnp.sum(h * l0_ref[...], axis=1, keepdims=True) + l0b_ref[...]
    m = h + a0_ref[...] + a1_ref[...]
    y = jnp.dot(m, wc_ref[...], preferred_element_type=jnp.float32)
    y = y + bc_ref[...]
    mu = jnp.mean(y, axis=0, keepdims=True)
    var = jnp.mean((y - mu) ** 2, axis=0, keepdims=True)
    h2 = gc_ref[...] * (y - mu) * lax.rsqrt(var + 1e-5) + bec_ref[...]
    h2 = jnp.maximum(h2, 0.0)
    np1 = jnp.sum(h2 * l1_ref[...], axis=1, keepdims=True) + l1b_ref[...]
    node_ref[...] = np0 + np1
    ids = lax.broadcasted_iota(jnp.int32, (1, G), 1)
    mask = batch_ref[...] == ids
    neg = jnp.float32(-jnp.inf)
    w0 = jnp.max(jnp.where(mask, np0, neg), axis=0)
    w1 = jnp.max(jnp.where(mask, np1, neg), axis=0)
    wsi_ref[...] = (w0 + w1)[:, None]


def kernel(x, edge_index, batch, W1, b1, g1, be1, L0_W, L0_b,
           Wc, bc, gc, bec, L1_W, L1_b):
    f32 = jnp.float32
    W1p = jnp.pad(W1, ((0, 0), (0, HP - W1.shape[1])))
    b1p = jnp.pad(b1, (0, HP - b1.shape[0]))[None, :]
    g1p = jnp.pad(g1, (0, HP - g1.shape[0]))[None, :]
    be1p = jnp.pad(be1, (0, HP - be1.shape[0]))[None, :]
    Wcp = jnp.pad(Wc, ((0, HP - Wc.shape[0]), (0, HP - Wc.shape[1])))
    bcp = jnp.pad(bc, (0, HP - bc.shape[0]))[None, :]
    gcp = jnp.pad(gc, (0, HP - gc.shape[0]))[None, :]
    becp = jnp.pad(bec, (0, HP - bec.shape[0]))[None, :]
    l0 = jnp.pad(L0_W[:, 0], (0, HP - L0_W.shape[0]))[None, :]
    l1 = jnp.pad(L1_W[:, 0], (0, HP - L1_W.shape[0]))[None, :]
    l0b = L0_b.reshape(1, 1).astype(f32)
    l1b = L1_b.reshape(1, 1).astype(f32)

    h = pl.pallas_call(
        _first_layer_body,
        out_shape=jax.ShapeDtypeStruct((N, HP), f32),
    )(x, W1p, b1p, g1p, be1p)

    src = edge_index[0]
    dst = edge_index[1]
    hp = jnp.pad(h, ((0, NROW - N), (0, 0)))
    zeros = jnp.zeros((NROW, HP), f32)
    agg = _segment_sum_sc(hp, zeros, src, dst)

    wsi, node = pl.pallas_call(
        _second_layer_body,
        out_shape=(
            jax.ShapeDtypeStruct((G, 1), f32),
            jax.ShapeDtypeStruct((N, 1), f32),
        ),
    )(h, agg[0, :N], agg[1, :N], batch[:, None], Wcp, bcp, gcp, becp,
      l0, l0b, l1, l1b)
    return (wsi, node)
